# Initial kernel scaffold; baseline (speedup 1.0000x reference)
import jax
import jax.numpy as jnp
from jax.experimental import pallas as pl


def kernel(x, edge_index, edge_attr, W1, b1, W2, b2, W3, b3, Wl, bl):
    raise NotImplementedError("write your pallas kernel here")


# speedup vs baseline: 1.4987x; 1.4987x over previous
import jax
import jax.numpy as jnp
from jax import lax
from jax.experimental import pallas as pl
from jax.experimental.pallas import tpu as pltpu
from jax.experimental.pallas import tpu_sc as plsc

N = 10000
E = 640000
F = 1700
H = 64
HH = 16
NSPLIT = H // HH

NC = 2
NS = 16
NW = NC * NS
CH = 128
NCHUNK = 157
EPAD = NW * NCHUNK * CH
NPAD = 10240
NR = NPAD // NS
RB = 1000
LANES = 16


def _zero_f32(ref, nvec):
    def body(i, carry):
        ref[pl.ds(i * LANES, LANES)] = jnp.zeros((LANES,), jnp.float32)
        return carry
    lax.fori_loop(0, nvec, body, None)


def _sc_deg_body(dst_hbm, ew_hbm, deg_out, dstv, ewv, degv):
    c = lax.axis_index("c")
    s = lax.axis_index("s")
    wid = c * NS + s
    pltpu.sync_copy(dst_hbm.at[wid], dstv)
    pltpu.sync_copy(ew_hbm.at[wid], ewv)
    _zero_f32(degv, NPAD // LANES)

    def chunk(g, carry):
        for t in range(CH // LANES):
            idx = dstv[g, pl.ds(t * LANES, LANES)]
            val = ewv[g, pl.ds(t * LANES, LANES)]
            plsc.addupdate_scatter(degv, [idx], val)
        return carry
    lax.fori_loop(0, NCHUNK, chunk, None)
    pltpu.sync_copy(degv, deg_out.at[wid])


def _make_sc_msg(with_cpart):
    def body(src_hbm, dst_hbm, ew_hbm, *rest):
        xs_hbms = rest[:NSPLIT]
        rest = rest[NSPLIT:]
        if with_cpart:
            (dinv_hbm, z_hbm) = rest[:2]
            acc_outs = rest[2:2 + NSPLIT]
            (cp_out, srcv, dstv, ewv, rowsv, dinvv, cpv,
             xs_sp, acc_sp, sem) = rest[2 + NSPLIT:]
        else:
            (z_hbm,) = rest[:1]
            acc_outs = rest[1:1 + NSPLIT]
            (srcv, dstv, ewv, rowsv, xs_sp, acc_sp, sem) = rest[1 + NSPLIT:]
        c = lax.axis_index("c")
        s = lax.axis_index("s")
        wid = c * NS + s
        pltpu.sync_copy(src_hbm.at[wid], srcv)
        pltpu.sync_copy(dst_hbm.at[wid], dstv)
        pltpu.sync_copy(ew_hbm.at[wid], ewv)
        if with_cpart:
            pltpu.sync_copy(dinv_hbm, dinvv)
            _zero_f32(cpv, NPAD // LANES)

        for h in range(NSPLIT):
            xs_h = xs_hbms[h]

            @pl.when(s == 0)
            def _():
                pltpu.sync_copy(xs_h, xs_sp.at[pl.ds(0, N)])
            pltpu.sync_copy(z_hbm, acc_sp.at[pl.ds(s * NR, NR)])
            plsc.subcore_barrier()

            def chunk(g, carry):
                pltpu.async_copy(xs_sp.at[srcv.at[g]], rowsv, sem).wait()

                def mul(t, c2):
                    ev = ewv[g, pl.ds(t * LANES, LANES)]
                    for l in range(LANES):
                        w = ev[l]
                        row = t * LANES + l
                        rowsv[row, :] = rowsv[row, :] * w
                    return c2
                lax.fori_loop(0, CH // LANES, mul, None)
                pltpu.sync_copy(rowsv, acc_sp.at[dstv.at[g]], add=True)
                if with_cpart and h == 0:
                    for t in range(CH // LANES):
                        sl = pl.ds(t * LANES, LANES)
                        di = dstv[g, sl]
                        dv = plsc.load_gather(dinvv, [di])
                        ev = ewv[g, sl]
                        si = srcv[g, sl]
                        plsc.addupdate_scatter(cpv, [si], ev * dv)
                return carry
            lax.fori_loop(0, NCHUNK, chunk, None)
            plsc.subcore_barrier()
            pltpu.sync_copy(acc_sp.at[pl.ds(s * NR, NR)],
                            acc_outs[h].at[c, pl.ds(s * NR, NR)])
        if with_cpart:
            pltpu.sync_copy(cpv, cp_out.at[wid])
    return body


def _sc_msg(src3, dst3, ew3, xs_list, dinv1, zeros_nr, with_cpart):
    out_type = [jax.ShapeDtypeStruct((NC, NPAD, HH), jnp.float32)
                for _ in range(NSPLIT)]
    if with_cpart:
        out_type.append(jax.ShapeDtypeStruct((NW, NPAD), jnp.float32))
    scratch = [
        pltpu.VMEM((NCHUNK, CH), jnp.int32),
        pltpu.VMEM((NCHUNK, CH), jnp.int32),
        pltpu.VMEM((NCHUNK, CH), jnp.float32),
        pltpu.VMEM((CH, HH), jnp.float32),
    ]
    if with_cpart:
        scratch += [
            pltpu.VMEM((N,), jnp.float32),
            pltpu.VMEM((NPAD,), jnp.float32),
        ]
    scratch += [
        pltpu.VMEM_SHARED((NPAD, HH), jnp.float32),
        pltpu.VMEM_SHARED((NPAD, HH), jnp.float32),
        pltpu.SemaphoreType.DMA,
    ]
    fn = pl.kernel(
        _make_sc_msg(with_cpart),
        out_type=out_type,
        mesh=plsc.VectorSubcoreMesh(core_axis_name="c", subcore_axis_name="s", num_cores=NC, num_subcores=NS),
        scratch_types=scratch,
        compiler_params=pltpu.CompilerParams(needs_layout_passes=False),
    )
    args = [src3, dst3, ew3, *xs_list]
    if with_cpart:
        args.append(dinv1)
    args.append(zeros_nr)
    return fn(*args)


def _tc1_body(x_ref, w1_ref, degp_ref, *out_refs):
    xs_refs = out_refs[:NSPLIT]
    dinv_ref = out_refs[NSPLIT]
    deg = jnp.sum(degp_ref[...], axis=0) + 1.0
    dinv = lax.rsqrt(deg)
    xw = jnp.dot(x_ref[...], w1_ref[...], preferred_element_type=jnp.float32)
    xs = dinv * xw
    for k in range(NSPLIT):
        xs_refs[k][...] = xs[:, k * HH:(k + 1) * HH]
    dinv_ref[...] = dinv


def _tc2_body(*refs):
    acc_refs = refs[:NSPLIT]
    xs_refs = refs[NSPLIT:2 * NSPLIT]
    dinv_ref, b1_ref, w2_ref = refs[2 * NSPLIT:2 * NSPLIT + 3]
    out_refs = refs[2 * NSPLIT + 3:]
    acc = jnp.concatenate([r[0] + r[1] for r in acc_refs], axis=1)
    xs1 = jnp.concatenate([r[...] for r in xs_refs], axis=1)
    h1 = jnp.maximum(dinv_ref[...] * (acc + xs1) + b1_ref[...], 0.0)
    xs2 = dinv_ref[...] * jnp.dot(
        h1, w2_ref[...], preferred_element_type=jnp.float32)
    for k in range(NSPLIT):
        out_refs[k][...] = xs2[:, k * HH:(k + 1) * HH]


def _tc3_body(*refs):
    acc_refs = refs[:NSPLIT]
    xs_refs = refs[NSPLIT:2 * NSPLIT]
    (dinv_ref, cp_ref, b2_ref, w3_ref, b3_ref, wl_ref, bl_ref,
     out_ref, s_acc) = refs[2 * NSPLIT:]
    i = pl.program_id(0)

    @pl.when(i == 0)
    def _():
        s_acc[...] = jnp.zeros_like(s_acc)

    acc = jnp.concatenate([r[0] + r[1] for r in acc_refs], axis=1)
    xs2 = jnp.concatenate([r[...] for r in xs_refs], axis=1)
    h2 = jnp.maximum(dinv_ref[...] * (acc + xs2) + b2_ref[...], 0.0)
    cp = jnp.sum(cp_ref[...], axis=0)
    cvec = dinv_ref[...] * cp + dinv_ref[...] * dinv_ref[...]
    s_acc[...] += jnp.sum(cvec * h2, axis=0, keepdims=True)

    @pl.when(i == pl.num_programs(0) - 1)
    def _():
        pooled = jnp.dot(s_acc[...] * (1.0 / N), w3_ref[...],
                         preferred_element_type=jnp.float32) + b3_ref[...]
        out_ref[...] = jnp.dot(pooled, wl_ref[...],
                               preferred_element_type=jnp.float32) + bl_ref[...]


_SPEC_Q = pl.BlockSpec((RB, HH), lambda i: (i, 0))
_SPEC_ACC = pl.BlockSpec((NC, RB, HH), lambda i: (0, i, 0))
_SPEC_DINV = pl.BlockSpec((RB, 1), lambda i: (i, 0))
_TYPE_Q = jax.ShapeDtypeStruct((N, HH), jnp.float32)


def kernel(x, edge_index, edge_attr, W1, b1, W2, b2, W3, b3, Wl, bl):
    src = edge_index[0]
    dst = edge_index[1]
    pad = EPAD - E
    src3 = jnp.concatenate([src, jnp.zeros((pad,), jnp.int32)]).reshape(NW, NCHUNK, CH)
    dst3 = jnp.concatenate([dst, jnp.zeros((pad,), jnp.int32)]).reshape(NW, NCHUNK, CH)
    ew3 = jnp.concatenate([edge_attr, jnp.zeros((pad,), jnp.float32)]).reshape(NW, NCHUNK, CH)
    zeros_nr = jnp.zeros((NR, HH), jnp.float32)

    deg_parts = pl.kernel(
        _sc_deg_body,
        out_type=jax.ShapeDtypeStruct((NW, NPAD), jnp.float32),
        mesh=plsc.VectorSubcoreMesh(core_axis_name="c", subcore_axis_name="s", num_cores=NC, num_subcores=NS),
        scratch_types=[
            pltpu.VMEM((NCHUNK, CH), jnp.int32),
            pltpu.VMEM((NCHUNK, CH), jnp.float32),
            pltpu.VMEM((NPAD,), jnp.float32),
        ],
        compiler_params=pltpu.CompilerParams(needs_layout_passes=False),
    )(dst3, ew3)

    *xs1_list, dinv = pl.pallas_call(
        _tc1_body,
        grid=(N // RB,),
        in_specs=[
            pl.BlockSpec((RB, F), lambda i: (i, 0)),
            pl.BlockSpec((F, H), lambda i: (0, 0)),
            pl.BlockSpec((NW, RB, 1), lambda i: (0, i, 0)),
        ],
        out_specs=[_SPEC_Q] * NSPLIT + [_SPEC_DINV],
        out_shape=[_TYPE_Q] * NSPLIT + [jax.ShapeDtypeStruct((N, 1), jnp.float32)],
    )(x, W1, deg_parts.reshape(NW, NPAD, 1))

    dinv1 = dinv.reshape(N)

    def _jnp_acc(xs_list_):
        xs_full = jnp.concatenate(xs_list_, axis=1)
        acc = jnp.zeros((N, H), jnp.float32).at[dst].add(
            edge_attr[:, None] * xs_full[src])
        accp = jnp.pad(acc, ((0, NPAD - N), (0, 0)))
        return [jnp.stack([accp[:, k * HH:(k + 1) * HH],
                           jnp.zeros((NPAD, HH), jnp.float32)])
                for k in range(NSPLIT)]

    acc1_list = _jnp_acc(xs1_list)
    cpart1 = jnp.zeros((N,), jnp.float32).at[src].add(edge_attr * dinv1[dst])
    cpart = jnp.pad(cpart1, (0, NPAD - N))[None, :] * jnp.ones(
        (NW, 1), jnp.float32) / NW

    xs2_list = pl.pallas_call(
        _tc2_body,
        grid=(N // RB,),
        in_specs=[_SPEC_ACC] * NSPLIT + [_SPEC_Q] * NSPLIT + [
            _SPEC_DINV,
            pl.BlockSpec((1, H), lambda i: (0, 0)),
            pl.BlockSpec((H, H), lambda i: (0, 0)),
        ],
        out_specs=[_SPEC_Q] * NSPLIT,
        out_shape=[_TYPE_Q] * NSPLIT,
    )(*acc1_list, *xs1_list, dinv, b1.reshape(1, H), W2)

    acc2_list = _jnp_acc(xs2_list)

    out = pl.pallas_call(
        _tc3_body,
        grid=(N // RB,),
        in_specs=[_SPEC_ACC] * NSPLIT + [_SPEC_Q] * NSPLIT + [
            _SPEC_DINV,
            pl.BlockSpec((NW, RB, 1), lambda i: (0, i, 0)),
            pl.BlockSpec((1, H), lambda i: (0, 0)),
            pl.BlockSpec((H, H), lambda i: (0, 0)),
            pl.BlockSpec((1, H), lambda i: (0, 0)),
            pl.BlockSpec((H, F), lambda i: (0, 0)),
            pl.BlockSpec((1, F), lambda i: (0, 0)),
        ],
        out_specs=pl.BlockSpec((1, F), lambda i: (0, 0)),
        out_shape=jax.ShapeDtypeStruct((1, F), jnp.float32),
        scratch_shapes=[pltpu.VMEM((1, H), jnp.float32)],
    )(*acc2_list, *xs2_list, dinv, cpart.reshape(NW, NPAD, 1),
      b2.reshape(1, H), W3, b3.reshape(1, H), Wl, bl.reshape(1, F))

    return out


# speedup vs baseline: 2.7284x; 1.8206x over previous
import jax
import jax.numpy as jnp
from jax import lax
from jax.experimental import pallas as pl
from jax.experimental.pallas import tpu as pltpu
from jax.experimental.pallas import tpu_sc as plsc

N = 10000
E = 640000
F = 1700
H = 64
HH = 16
NSPLIT = H // HH

NC = 2
NS = 16
NW = NC * NS
CH = 128
NCHUNK = 157
EPAD = NW * NCHUNK * CH
NPAD = 10240
NR = NPAD // NS
RB = 1000
LANES = 16


def _zero_f32(ref, nvec):
    def body(i, carry):
        ref[pl.ds(i * LANES, LANES)] = jnp.zeros((LANES,), jnp.float32)
        return carry
    lax.fori_loop(0, nvec, body, None)


def _sc_deg_body(dst_hbm, ew_hbm, deg_out, dstv, ewv, degv):
    c = lax.axis_index("c")
    s = lax.axis_index("s")
    wid = c * NS + s
    pltpu.sync_copy(dst_hbm.at[wid], dstv)
    pltpu.sync_copy(ew_hbm.at[wid], ewv)
    _zero_f32(degv, NPAD // LANES)

    def chunk(g, carry):
        for t in range(CH // LANES):
            idx = dstv[g, pl.ds(t * LANES, LANES)]
            val = ewv[g, pl.ds(t * LANES, LANES)]
            plsc.addupdate_scatter(degv, [idx], val)
        return carry
    lax.fori_loop(0, NCHUNK, chunk, None)
    pltpu.sync_copy(degv, deg_out.at[wid])


def _make_sc_msg(with_cpart):
    def body(src_hbm, dst_hbm, ew_hbm, *rest):
        xs_hbms = rest[:NSPLIT]
        rest = rest[NSPLIT:]
        if with_cpart:
            (dinv_hbm, z_hbm) = rest[:2]
            acc_outs = rest[2:2 + NSPLIT]
            (cp_out, srcv, dstv, ewv, rowsv, dinvv, cpv,
             xs_sp, acc_sp, sem) = rest[2 + NSPLIT:]
        else:
            (z_hbm,) = rest[:1]
            acc_outs = rest[1:1 + NSPLIT]
            (srcv, dstv, ewv, rowsv, xs_sp, acc_sp, sem) = rest[1 + NSPLIT:]
        c = lax.axis_index("c")
        s = lax.axis_index("s")
        wid = c * NS + s
        pltpu.sync_copy(src_hbm.at[wid], srcv)
        pltpu.sync_copy(dst_hbm.at[wid], dstv)
        pltpu.sync_copy(ew_hbm.at[wid], ewv)
        if with_cpart:
            pltpu.sync_copy(dinv_hbm, dinvv)
            _zero_f32(cpv, NPAD // LANES)

        for h in range(NSPLIT):
            xs_h = xs_hbms[h]

            @pl.when(s == 0)
            def _():
                pltpu.sync_copy(xs_h, xs_sp.at[pl.ds(0, N)])
            pltpu.sync_copy(z_hbm, acc_sp.at[pl.ds(s * NR, NR)])
            plsc.subcore_barrier()

            def chunk(g, carry):
                pltpu.async_copy(xs_sp.at[srcv.at[g]], rowsv, sem).wait()

                def mul(t, c2):
                    ev = ewv[g, pl.ds(t * LANES, LANES)]
                    for l in range(LANES):
                        w = ev[l]
                        row = t * LANES + l
                        rowsv[row, :] = rowsv[row, :] * w
                    return c2
                lax.fori_loop(0, CH // LANES, mul, None)
                pltpu.sync_copy(rowsv, acc_sp.at[dstv.at[g]], add=True)
                if with_cpart and h == 0:
                    for t in range(CH // LANES):
                        sl = pl.ds(t * LANES, LANES)
                        di = dstv[g, sl]
                        dv = plsc.load_gather(dinvv, [di])
                        ev = ewv[g, sl]
                        si = srcv[g, sl]
                        plsc.addupdate_scatter(cpv, [si], ev * dv)
                return carry
            lax.fori_loop(0, NCHUNK, chunk, None)
            plsc.subcore_barrier()
            pltpu.sync_copy(acc_sp.at[pl.ds(s * NR, NR)],
                            acc_outs[h].at[c, pl.ds(s * NR, NR)])
        if with_cpart:
            pltpu.sync_copy(cpv, cp_out.at[wid])
    return body


def _sc_msg(src3, dst3, ew3, xs_list, dinv1, zeros_nr, with_cpart):
    out_type = [jax.ShapeDtypeStruct((NC, NPAD, HH), jnp.float32)
                for _ in range(NSPLIT)]
    if with_cpart:
        out_type.append(jax.ShapeDtypeStruct((NW, NPAD), jnp.float32))
    scratch = [
        pltpu.VMEM((NCHUNK, CH), jnp.int32),
        pltpu.VMEM((NCHUNK, CH), jnp.int32),
        pltpu.VMEM((NCHUNK, CH), jnp.float32),
        pltpu.VMEM((CH, HH), jnp.float32),
    ]
    if with_cpart:
        scratch += [
            pltpu.VMEM((N,), jnp.float32),
            pltpu.VMEM((NPAD,), jnp.float32),
        ]
    scratch += [
        pltpu.VMEM_SHARED((NPAD, HH), jnp.float32),
        pltpu.VMEM_SHARED((NPAD, HH), jnp.float32),
        pltpu.SemaphoreType.DMA,
    ]
    fn = pl.kernel(
        _make_sc_msg(with_cpart),
        out_type=out_type,
        mesh=plsc.VectorSubcoreMesh(core_axis_name="c", subcore_axis_name="s", num_cores=NC, num_subcores=NS),
        scratch_types=scratch,
        compiler_params=pltpu.CompilerParams(needs_layout_passes=False),
    )
    args = [src3, dst3, ew3, *xs_list]
    if with_cpart:
        args.append(dinv1)
    args.append(zeros_nr)
    return fn(*args)


def _sc_cpart_body(src_hbm, dst_hbm, ew_hbm, dinv_hbm, cp_out,
                   srcv, dstv, ewv, dinvv, cpv):
    c = lax.axis_index("c")
    s = lax.axis_index("s")
    wid = c * NS + s
    pltpu.sync_copy(src_hbm.at[wid], srcv)
    pltpu.sync_copy(dst_hbm.at[wid], dstv)
    pltpu.sync_copy(ew_hbm.at[wid], ewv)
    pltpu.sync_copy(dinv_hbm, dinvv)
    _zero_f32(cpv, NPAD // LANES)

    def chunk(g, carry):
        for t in range(CH // LANES):
            sl = pl.ds(t * LANES, LANES)
            di = dstv[g, sl]
            dv = plsc.load_gather(dinvv, [di])
            ev = ewv[g, sl]
            si = srcv[g, sl]
            plsc.addupdate_scatter(cpv, [si], ev * dv)
        return carry
    lax.fori_loop(0, NCHUNK, chunk, None)
    pltpu.sync_copy(cpv, cp_out.at[wid])


def _sc_cpart(src3, dst3, ew3, dinv1):
    fn = pl.kernel(
        _sc_cpart_body,
        out_type=jax.ShapeDtypeStruct((NW, NPAD), jnp.float32),
        mesh=plsc.VectorSubcoreMesh(core_axis_name="c", subcore_axis_name="s",
                                    num_cores=NC, num_subcores=NS),
        scratch_types=[
            pltpu.VMEM((NCHUNK, CH), jnp.int32),
            pltpu.VMEM((NCHUNK, CH), jnp.int32),
            pltpu.VMEM((NCHUNK, CH), jnp.float32),
            pltpu.VMEM((N,), jnp.float32),
            pltpu.VMEM((NPAD,), jnp.float32),
        ],
        compiler_params=pltpu.CompilerParams(needs_layout_passes=False),
    )
    return fn(src3, dst3, ew3, dinv1)


def _tc1_body(x_ref, w1_ref, degp_ref, *out_refs):
    xs_refs = out_refs[:NSPLIT]
    dinv_ref = out_refs[NSPLIT]
    deg = jnp.sum(degp_ref[...], axis=0) + 1.0
    dinv = lax.rsqrt(deg)
    xw = jnp.dot(x_ref[...], w1_ref[...], preferred_element_type=jnp.float32)
    xs = dinv * xw
    for k in range(NSPLIT):
        xs_refs[k][...] = xs[:, k * HH:(k + 1) * HH]
    dinv_ref[...] = dinv


def _tc2_body(*refs):
    acc_refs = refs[:NSPLIT]
    xs_refs = refs[NSPLIT:2 * NSPLIT]
    dinv_ref, b1_ref, w2_ref = refs[2 * NSPLIT:2 * NSPLIT + 3]
    out_refs = refs[2 * NSPLIT + 3:]
    acc = jnp.concatenate([r[0] + r[1] for r in acc_refs], axis=1)
    xs1 = jnp.concatenate([r[...] for r in xs_refs], axis=1)
    h1 = jnp.maximum(dinv_ref[...] * (acc + xs1) + b1_ref[...], 0.0)
    xs2 = dinv_ref[...] * jnp.dot(
        h1, w2_ref[...], preferred_element_type=jnp.float32)
    for k in range(NSPLIT):
        out_refs[k][...] = xs2[:, k * HH:(k + 1) * HH]


def _tc3_body(*refs):
    acc_refs = refs[:NSPLIT]
    xs_refs = refs[NSPLIT:2 * NSPLIT]
    (dinv_ref, cp_ref, b2_ref, w3_ref, b3_ref, wl_ref, bl_ref,
     out_ref, s_acc) = refs[2 * NSPLIT:]
    i = pl.program_id(0)

    @pl.when(i == 0)
    def _():
        s_acc[...] = jnp.zeros_like(s_acc)

    acc = jnp.concatenate([r[0] + r[1] for r in acc_refs], axis=1)
    xs2 = jnp.concatenate([r[...] for r in xs_refs], axis=1)
    h2 = jnp.maximum(dinv_ref[...] * (acc + xs2) + b2_ref[...], 0.0)
    cp = jnp.sum(cp_ref[...], axis=0)
    cvec = dinv_ref[...] * cp + dinv_ref[...] * dinv_ref[...]
    s_acc[...] += jnp.sum(cvec * h2, axis=0, keepdims=True)

    @pl.when(i == pl.num_programs(0) - 1)
    def _():
        pooled = jnp.dot(s_acc[...] * (1.0 / N), w3_ref[...],
                         preferred_element_type=jnp.float32) + b3_ref[...]
        out_ref[...] = jnp.dot(pooled, wl_ref[...],
                               preferred_element_type=jnp.float32) + bl_ref[...]


_SPEC_Q = pl.BlockSpec((RB, HH), lambda i: (i, 0))
_SPEC_ACC = pl.BlockSpec((NC, RB, HH), lambda i: (0, i, 0))
_SPEC_DINV = pl.BlockSpec((RB, 1), lambda i: (i, 0))
_TYPE_Q = jax.ShapeDtypeStruct((N, HH), jnp.float32)


def kernel(x, edge_index, edge_attr, W1, b1, W2, b2, W3, b3, Wl, bl):
    src = edge_index[0]
    dst = edge_index[1]
    pad = EPAD - E
    src3 = jnp.concatenate([src, jnp.zeros((pad,), jnp.int32)]).reshape(NW, NCHUNK, CH)
    dst3 = jnp.concatenate([dst, jnp.zeros((pad,), jnp.int32)]).reshape(NW, NCHUNK, CH)
    ew3 = jnp.concatenate([edge_attr, jnp.zeros((pad,), jnp.float32)]).reshape(NW, NCHUNK, CH)
    zeros_nr = jnp.zeros((NR, HH), jnp.float32)

    deg_parts = pl.kernel(
        _sc_deg_body,
        out_type=jax.ShapeDtypeStruct((NW, NPAD), jnp.float32),
        mesh=plsc.VectorSubcoreMesh(core_axis_name="c", subcore_axis_name="s", num_cores=NC, num_subcores=NS),
        scratch_types=[
            pltpu.VMEM((NCHUNK, CH), jnp.int32),
            pltpu.VMEM((NCHUNK, CH), jnp.float32),
            pltpu.VMEM((NPAD,), jnp.float32),
        ],
        compiler_params=pltpu.CompilerParams(needs_layout_passes=False),
    )(dst3, ew3)

    *xs1_list, dinv = pl.pallas_call(
        _tc1_body,
        grid=(N // RB,),
        in_specs=[
            pl.BlockSpec((RB, F), lambda i: (i, 0)),
            pl.BlockSpec((F, H), lambda i: (0, 0)),
            pl.BlockSpec((NW, RB, 1), lambda i: (0, i, 0)),
        ],
        out_specs=[_SPEC_Q] * NSPLIT + [_SPEC_DINV],
        out_shape=[_TYPE_Q] * NSPLIT + [jax.ShapeDtypeStruct((N, 1), jnp.float32)],
    )(x, W1, deg_parts.reshape(NW, NPAD, 1))

    dinv1 = dinv.reshape(N)

    def _jnp_acc(xs_list_):
        xs_full = jnp.concatenate(xs_list_, axis=1)
        acc = jnp.zeros((N, H), jnp.float32).at[dst].add(
            edge_attr[:, None] * xs_full[src])
        accp = jnp.pad(acc, ((0, NPAD - N), (0, 0)))
        return [jnp.stack([accp[:, k * HH:(k + 1) * HH],
                           jnp.zeros((NPAD, HH), jnp.float32)])
                for k in range(NSPLIT)]

    acc1_list = _jnp_acc(xs1_list)
    cpart = _sc_cpart(src3, dst3, ew3, dinv1)

    xs2_list = pl.pallas_call(
        _tc2_body,
        grid=(N // RB,),
        in_specs=[_SPEC_ACC] * NSPLIT + [_SPEC_Q] * NSPLIT + [
            _SPEC_DINV,
            pl.BlockSpec((1, H), lambda i: (0, 0)),
            pl.BlockSpec((H, H), lambda i: (0, 0)),
        ],
        out_specs=[_SPEC_Q] * NSPLIT,
        out_shape=[_TYPE_Q] * NSPLIT,
    )(*acc1_list, *xs1_list, dinv, b1.reshape(1, H), W2)

    acc2_list = _jnp_acc(xs2_list)

    out = pl.pallas_call(
        _tc3_body,
        grid=(N // RB,),
        in_specs=[_SPEC_ACC] * NSPLIT + [_SPEC_Q] * NSPLIT + [
            _SPEC_DINV,
            pl.BlockSpec((NW, RB, 1), lambda i: (0, i, 0)),
            pl.BlockSpec((1, H), lambda i: (0, 0)),
            pl.BlockSpec((H, H), lambda i: (0, 0)),
            pl.BlockSpec((1, H), lambda i: (0, 0)),
            pl.BlockSpec((H, F), lambda i: (0, 0)),
            pl.BlockSpec((1, F), lambda i: (0, 0)),
        ],
        out_specs=pl.BlockSpec((1, F), lambda i: (0, 0)),
        out_shape=jax.ShapeDtypeStruct((1, F), jnp.float32),
        scratch_shapes=[pltpu.VMEM((1, H), jnp.float32)],
    )(*acc2_list, *xs2_list, dinv, cpart.reshape(NW, NPAD, 1),
      b2.reshape(1, H), W3, b3.reshape(1, H), Wl, bl.reshape(1, F))

    return out


# speedup vs baseline: 2.7291x; 1.0002x over previous
import jax
import jax.numpy as jnp
from jax import lax
from jax.experimental import pallas as pl
from jax.experimental.pallas import tpu as pltpu
from jax.experimental.pallas import tpu_sc as plsc

N = 10000
E = 640000
F = 1700
H = 64
HH = 16
NSPLIT = H // HH

NC = 2
NS = 16
NW = NC * NS
CH = 128
NCHUNK = 157
EPAD = NW * NCHUNK * CH
NPAD = 10240
NR = NPAD // NS
RB = 1000
LANES = 16


def _zero_f32(ref, nvec):
    def body(i, carry):
        ref[pl.ds(i * LANES, LANES)] = jnp.zeros((LANES,), jnp.float32)
        return carry
    lax.fori_loop(0, nvec, body, None)


def _sc_deg_body(dst_hbm, ew_hbm, deg_out, dstv, ewv, degv):
    c = lax.axis_index("c")
    s = lax.axis_index("s")
    wid = c * NS + s
    pltpu.sync_copy(dst_hbm.at[wid], dstv)
    pltpu.sync_copy(ew_hbm.at[wid], ewv)
    _zero_f32(degv, NPAD // LANES)

    def chunk(g, carry):
        for t in range(CH // LANES):
            idx = dstv[g, pl.ds(t * LANES, LANES)]
            val = ewv[g, pl.ds(t * LANES, LANES)]
            plsc.addupdate_scatter(degv, [idx], val)
        return carry
    lax.fori_loop(0, NCHUNK, chunk, None)
    pltpu.sync_copy(degv, deg_out.at[wid])


def _sc_cpart_body(src_hbm, dst_hbm, ew_hbm, dinv_hbm, cp_out,
                   srcv, dstv, ewv, dinvv, cpv):
    c = lax.axis_index("c")
    s = lax.axis_index("s")
    wid = c * NS + s
    pltpu.sync_copy(src_hbm.at[wid], srcv)
    pltpu.sync_copy(dst_hbm.at[wid], dstv)
    pltpu.sync_copy(ew_hbm.at[wid], ewv)
    pltpu.sync_copy(dinv_hbm, dinvv)
    _zero_f32(cpv, NPAD // LANES)

    def chunk(g, carry):
        for t in range(CH // LANES):
            sl = pl.ds(t * LANES, LANES)
            di = dstv[g, sl]
            dv = plsc.load_gather(dinvv, [di])
            ev = ewv[g, sl]
            si = srcv[g, sl]
            plsc.addupdate_scatter(cpv, [si], ev * dv)
        return carry
    lax.fori_loop(0, NCHUNK, chunk, None)
    pltpu.sync_copy(cpv, cp_out.at[wid])


def _sc_cpart(src3, dst3, ew3, dinv1):
    fn = pl.kernel(
        _sc_cpart_body,
        out_type=jax.ShapeDtypeStruct((NW, NPAD), jnp.float32),
        mesh=plsc.VectorSubcoreMesh(core_axis_name="c", subcore_axis_name="s",
                                    num_cores=NC, num_subcores=NS),
        scratch_types=[
            pltpu.VMEM((NCHUNK, CH), jnp.int32),
            pltpu.VMEM((NCHUNK, CH), jnp.int32),
            pltpu.VMEM((NCHUNK, CH), jnp.float32),
            pltpu.VMEM((N,), jnp.float32),
            pltpu.VMEM((NPAD,), jnp.float32),
        ],
        compiler_params=pltpu.CompilerParams(needs_layout_passes=False),
    )
    return fn(src3, dst3, ew3, dinv1)


def _tc1_body(x_ref, w1_ref, degp_ref, *out_refs):
    xs_refs = out_refs[:NSPLIT]
    dinv_ref = out_refs[NSPLIT]
    deg = jnp.sum(degp_ref[...], axis=0) + 1.0
    dinv = lax.rsqrt(deg)
    xw = jnp.dot(x_ref[...], w1_ref[...], preferred_element_type=jnp.float32)
    xs = dinv * xw
    for k in range(NSPLIT):
        xs_refs[k][...] = xs[:, k * HH:(k + 1) * HH]
    dinv_ref[...] = dinv


def _tc2_body(*refs):
    acc_refs = refs[:NSPLIT]
    xs_refs = refs[NSPLIT:2 * NSPLIT]
    dinv_ref, b1_ref, w2_ref = refs[2 * NSPLIT:2 * NSPLIT + 3]
    out_refs = refs[2 * NSPLIT + 3:]
    acc = jnp.concatenate([r[0] + r[1] for r in acc_refs], axis=1)
    xs1 = jnp.concatenate([r[...] for r in xs_refs], axis=1)
    h1 = jnp.maximum(dinv_ref[...] * (acc + xs1) + b1_ref[...], 0.0)
    xs2 = dinv_ref[...] * jnp.dot(
        h1, w2_ref[...], preferred_element_type=jnp.float32)
    for k in range(NSPLIT):
        out_refs[k][...] = xs2[:, k * HH:(k + 1) * HH]


def _tc3_body(*refs):
    acc_refs = refs[:NSPLIT]
    xs_refs = refs[NSPLIT:2 * NSPLIT]
    (dinv_ref, cp_ref, b2_ref, w3_ref, b3_ref, wl_ref, bl_ref,
     out_ref, s_acc) = refs[2 * NSPLIT:]
    i = pl.program_id(0)

    @pl.when(i == 0)
    def _():
        s_acc[...] = jnp.zeros_like(s_acc)

    acc = jnp.concatenate([r[0] + r[1] for r in acc_refs], axis=1)
    xs2 = jnp.concatenate([r[...] for r in xs_refs], axis=1)
    h2 = jnp.maximum(dinv_ref[...] * (acc + xs2) + b2_ref[...], 0.0)
    cp = jnp.sum(cp_ref[...], axis=0)
    cvec = dinv_ref[...] * cp + dinv_ref[...] * dinv_ref[...]
    s_acc[...] += jnp.sum(cvec * h2, axis=0, keepdims=True)

    @pl.when(i == pl.num_programs(0) - 1)
    def _():
        pooled = jnp.dot(s_acc[...] * (1.0 / N), w3_ref[...],
                         preferred_element_type=jnp.float32) + b3_ref[...]
        out_ref[...] = jnp.dot(pooled, wl_ref[...],
                               preferred_element_type=jnp.float32) + bl_ref[...]


_SPEC_Q = pl.BlockSpec((RB, HH), lambda i: (i, 0))
_SPEC_ACC = pl.BlockSpec((NC, RB, HH), lambda i: (0, i, 0))
_SPEC_DINV = pl.BlockSpec((RB, 1), lambda i: (i, 0))
_TYPE_Q = jax.ShapeDtypeStruct((N, HH), jnp.float32)


def kernel(x, edge_index, edge_attr, W1, b1, W2, b2, W3, b3, Wl, bl):
    src = edge_index[0]
    dst = edge_index[1]
    pad = EPAD - E
    src3 = jnp.concatenate([src, jnp.zeros((pad,), jnp.int32)]).reshape(NW, NCHUNK, CH)
    dst3 = jnp.concatenate([dst, jnp.zeros((pad,), jnp.int32)]).reshape(NW, NCHUNK, CH)
    ew3 = jnp.concatenate([edge_attr, jnp.zeros((pad,), jnp.float32)]).reshape(NW, NCHUNK, CH)
    zeros_nr = jnp.zeros((NR, HH), jnp.float32)

    deg_parts = pl.kernel(
        _sc_deg_body,
        out_type=jax.ShapeDtypeStruct((NW, NPAD), jnp.float32),
        mesh=plsc.VectorSubcoreMesh(core_axis_name="c", subcore_axis_name="s", num_cores=NC, num_subcores=NS),
        scratch_types=[
            pltpu.VMEM((NCHUNK, CH), jnp.int32),
            pltpu.VMEM((NCHUNK, CH), jnp.float32),
            pltpu.VMEM((NPAD,), jnp.float32),
        ],
        compiler_params=pltpu.CompilerParams(needs_layout_passes=False),
    )(dst3, ew3)

    *xs1_list, dinv = pl.pallas_call(
        _tc1_body,
        grid=(N // RB,),
        in_specs=[
            pl.BlockSpec((RB, F), lambda i: (i, 0)),
            pl.BlockSpec((F, H), lambda i: (0, 0)),
            pl.BlockSpec((NW, RB, 1), lambda i: (0, i, 0)),
        ],
        out_specs=[_SPEC_Q] * NSPLIT + [_SPEC_DINV],
        out_shape=[_TYPE_Q] * NSPLIT + [jax.ShapeDtypeStruct((N, 1), jnp.float32)],
    )(x, W1, deg_parts.reshape(NW, NPAD, 1))

    dinv1 = dinv.reshape(N)

    def _jnp_acc(xs_list_):
        xs_full = jnp.concatenate(xs_list_, axis=1)
        acc = jnp.zeros((N, H), jnp.float32).at[dst].add(
            edge_attr[:, None] * xs_full[src])
        accp = jnp.pad(acc, ((0, NPAD - N), (0, 0)))
        return [jnp.stack([accp[:, k * HH:(k + 1) * HH],
                           jnp.zeros((NPAD, HH), jnp.float32)])
                for k in range(NSPLIT)]

    acc1_list = _jnp_acc(xs1_list)
    cpart = _sc_cpart(src3, dst3, ew3, dinv1)

    xs2_list = pl.pallas_call(
        _tc2_body,
        grid=(N // RB,),
        in_specs=[_SPEC_ACC] * NSPLIT + [_SPEC_Q] * NSPLIT + [
            _SPEC_DINV,
            pl.BlockSpec((1, H), lambda i: (0, 0)),
            pl.BlockSpec((H, H), lambda i: (0, 0)),
        ],
        out_specs=[_SPEC_Q] * NSPLIT,
        out_shape=[_TYPE_Q] * NSPLIT,
    )(*acc1_list, *xs1_list, dinv, b1.reshape(1, H), W2)

    acc2_list = _jnp_acc(xs2_list)

    out = pl.pallas_call(
        _tc3_body,
        grid=(N // RB,),
        in_specs=[_SPEC_ACC] * NSPLIT + [_SPEC_Q] * NSPLIT + [
            _SPEC_DINV,
            pl.BlockSpec((NW, RB, 1), lambda i: (0, i, 0)),
            pl.BlockSpec((1, H), lambda i: (0, 0)),
            pl.BlockSpec((H, H), lambda i: (0, 0)),
            pl.BlockSpec((1, H), lambda i: (0, 0)),
            pl.BlockSpec((H, F), lambda i: (0, 0)),
            pl.BlockSpec((1, F), lambda i: (0, 0)),
        ],
        out_specs=pl.BlockSpec((1, F), lambda i: (0, 0)),
        out_shape=jax.ShapeDtypeStruct((1, F), jnp.float32),
        scratch_shapes=[pltpu.VMEM((1, H), jnp.float32)],
    )(*acc2_list, *xs2_list, dinv, cpart.reshape(NW, NPAD, 1),
      b2.reshape(1, H), W3, b3.reshape(1, H), Wl, bl.reshape(1, F))

    return out
